# baseline (speedup 1.0000x reference)
import functools

import jax
import jax.numpy as jnp
from jax import lax
from jax.experimental import pallas as pl
from jax.experimental.pallas import tpu as pltpu
from jax.experimental.pallas import tpu_sc as plsc

NUM_EMB = 1_000_000
DIM = 16
BATCH = 16384
BAG = 50

NUM_CORES = 2
NUM_SUBCORES = 16
NW = NUM_CORES * NUM_SUBCORES

BPW = BATCH // NW
IPW = BPW * BAG
GROUP_BAGS = 8
GROUP = GROUP_BAGS * BAG
SPLIT = 128
GPW = BPW // GROUP_BAGS
NBUF = 4

STRIDE = 16
CPT = 244 * 128
CC = 512
NCH = CPT // CC
TAIL0 = NW * CPT
TAIL1 = NUM_EMB - 128

_MESH = plsc.VectorSubcoreMesh(core_axis_name="c", subcore_axis_name="s")


@functools.partial(
    pl.kernel,
    mesh=_MESH,
    out_type=jax.ShapeDtypeStruct((NUM_EMB * STRIDE,), jnp.float32),
    compiler_params=pltpu.CompilerParams(use_tc_tiling_on_sc=True,
                                         needs_layout_passes=False),
    scratch_types=[
        pltpu.VMEM((CC // 128, DIM, 128), jnp.float32),
        pltpu.VMEM((CC // 128, DIM, 128), jnp.float32),
        pltpu.VMEM((CC * STRIDE,), jnp.float32),
        pltpu.VMEM((CC * STRIDE,), jnp.float32),
        pltpu.SemaphoreType.DMA,
        pltpu.SemaphoreType.DMA,
        pltpu.SemaphoreType.DMA,
        pltpu.SemaphoreType.DMA,
    ],
)
def _pack_sc(wt_hbm, t128_hbm, out_hbm, seg0, seg1, ov0, ov1,
             isem0, isem1, osem0, osem1):
    wid = lax.axis_index("s") * NUM_CORES + lax.axis_index("c")
    col0 = wid * CPT
    segs = (seg0, seg1)
    ovs = (ov0, ov1)
    isems = (isem0, isem1)
    osems = (osem0, osem1)
    lanes = lax.iota(jnp.int32, 16)
    lanes17 = lanes * STRIDE

    def in_copies(start, n, b):
        return [pltpu.make_async_copy(
            wt_hbm.at[:, pl.ds(start + 128 * t, 128)],
            segs[b].at[t], isems[b]) for t in range(n // 128)]

    def out_copy(start, n, b):
        return pltpu.make_async_copy(
            ovs[b].at[pl.ds(0, STRIDE * n)],
            out_hbm.at[pl.ds(STRIDE * start, STRIDE * n)], osems[b])

    def do_chunk(start, n, b):
        for c in in_copies(start, n, b):
            c.wait()
        seg, ov = segs[b], ovs[b]
        for t in range(n // 128):
            for sub in range(8):
                idx0 = lanes17 + (STRIDE * (128 * t + 16 * sub))
                for j in range(DIM):
                    v = seg[t, j, pl.ds(16 * sub, 16)]
                    plsc.store_scatter(ov, [idx0 + j], v)
        out_copy(start, n, b).start()

    for c in in_copies(col0, CC, 0) + in_copies(col0 + CC, CC, 1):
        c.start()

    def body(i, carry):
        for b in range(2):
            c = 2 * i + b
            @pl.when(c >= 2)
            def _():
                out_copy(col0 + CC * (c - 2), CC, b).wait()
            do_chunk(col0 + CC * c, CC, b)
            @pl.when(c + 2 < NCH)
            def _():
                for cp in in_copies(col0 + CC * (c + 2), CC, b):
                    cp.start()
        return carry

    lax.fori_loop(0, (NCH - 1) // 2, body, 0)

    is_last = wid == NW - 1

    @pl.when(is_last)
    def _():
        for cp in in_copies(TAIL0, 512, 1):
            cp.start()

    out_copy(col0 + CC * (NCH - 3), CC, 0).wait()
    do_chunk(col0 + CC * (NCH - 1), CC, 0)

    @pl.when(is_last)
    def _():
        pltpu.make_async_copy(t128_hbm, segs[0].at[0], isems[0]).start()
        out_copy(col0 + CC * (NCH - 2), CC, 1).wait()
        do_chunk(TAIL0, 512, 1)
        out_copy(col0 + CC * (NCH - 1), CC, 0).wait()
        pltpu.make_async_copy(t128_hbm, segs[0].at[0], isems[0]).wait()
        ov = ovs[0]
        for sub in range(8):
            idx0 = lanes17 + (STRIDE * 16 * sub)
            for j in range(DIM):
                v = seg0[0, j, pl.ds(16 * sub, 16)]
                plsc.store_scatter(ov, [idx0 + j], v)
        out_copy(TAIL1, 128, 0).start()
        out_copy(TAIL0, 512, 1).wait()
        out_copy(TAIL1, 128, 0).wait()

    @pl.when(jnp.logical_not(is_last))
    def _():
        out_copy(col0 + CC * (NCH - 2), CC, 1).wait()
        out_copy(col0 + CC * (NCH - 1), CC, 0).wait()


@functools.partial(
    pl.kernel,
    mesh=_MESH,
    out_type=jax.ShapeDtypeStruct((BATCH, DIM), jnp.float32),
    compiler_params=pltpu.CompilerParams(use_tc_tiling_on_sc=False),
    scratch_types=[
        pltpu.VMEM((IPW,), jnp.int32),
        pltpu.VMEM((BPW, DIM), jnp.float32),
    ] + [pltpu.VMEM((GROUP, STRIDE), jnp.float32) for _ in range(NBUF)]
      + [pltpu.SemaphoreType.DMA for _ in range(NBUF)],
)
def _embedding_bag_sc(idx_hbm, tbl_hbm, out_hbm, idx_v, out_v, *bufs):
    rows = bufs[:NBUF]
    sems = bufs[NBUF:]
    wid = lax.axis_index("s") * NUM_CORES + lax.axis_index("c")

    pltpu.sync_copy(idx_hbm.at[pl.ds(wid * IPW, IPW)], idx_v)

    def copies(g, b):
        base = GROUP * g
        cs = []
        off = 0
        while off < GROUP:
            n = min(SPLIT, GROUP - off)
            cs.append(pltpu.make_async_copy(
                tbl_hbm.at[idx_v.at[pl.ds(base + off, n)]],
                rows[b].at[pl.ds(off, n)], sems[b]))
            off += n
        return cs

    def start(g, b):
        for c in copies(g, b):
            c.start()

    def finish(g, b):
        for c in copies(g, b):
            c.wait()
        r = rows[b]
        for j in range(GROUP_BAGS):
            parts = []
            for c in range(5):
                base = BAG * j + 10 * c
                acc = r[base, pl.ds(0, DIM)]
                for k in range(base + 1, base + 10):
                    acc = acc + r[k, pl.ds(0, DIM)]
                parts.append(acc)
            total = (parts[0] + parts[1]) + (parts[2] + parts[3]) + parts[4]
            out_v[GROUP_BAGS * g + j] = total * jnp.float32(1.0 / BAG)

    for b in range(NBUF):
        start(b, b)

    def body(i, carry):
        for b in range(NBUF):
            g = NBUF * i + b
            finish(g, b)
            start(g + NBUF, b)
        return carry

    lax.fori_loop(0, GPW // NBUF - 1, body, 0)

    for b in range(NBUF):
        finish(GPW - NBUF + b, b)

    pltpu.sync_copy(out_v, out_hbm.at[pl.ds(wid * BPW, BPW)])


def kernel(input, weight):
    table = _pack_sc(weight.T, weight[NUM_EMB - 128:].T)
    table = table.reshape(NUM_EMB, STRIDE)
    return _embedding_bag_sc(input.astype(jnp.int32).reshape(-1), table)


# speedup vs baseline: 1.0735x; 1.0735x over previous
import functools

import jax
import jax.numpy as jnp
from jax import lax
from jax.experimental import pallas as pl
from jax.experimental.pallas import tpu as pltpu
from jax.experimental.pallas import tpu_sc as plsc

NUM_EMB = 1_000_000
DIM = 16
BATCH = 16384
BAG = 50

NUM_CORES = 2
NUM_SUBCORES = 16
NW = NUM_CORES * NUM_SUBCORES

BPW = BATCH // NW
IPW = BPW * BAG
GROUP_BAGS = 4
GROUP = GROUP_BAGS * BAG
SPLIT = 128
GPW = BPW // GROUP_BAGS
NBUF = 4

STRIDE = 16
CPT = 244 * 128
CC = 512
NCH = CPT // CC
TAIL0 = NW * CPT
TAIL1 = NUM_EMB - 128

_MESH = plsc.VectorSubcoreMesh(core_axis_name="c", subcore_axis_name="s")


@functools.partial(
    pl.kernel,
    mesh=_MESH,
    out_type=jax.ShapeDtypeStruct((NUM_EMB * STRIDE,), jnp.float32),
    compiler_params=pltpu.CompilerParams(use_tc_tiling_on_sc=True,
                                         needs_layout_passes=False),
    scratch_types=[
        pltpu.VMEM((CC // 128, DIM, 128), jnp.float32),
        pltpu.VMEM((CC // 128, DIM, 128), jnp.float32),
        pltpu.VMEM((CC * STRIDE,), jnp.float32),
        pltpu.VMEM((CC * STRIDE,), jnp.float32),
        pltpu.SemaphoreType.DMA,
        pltpu.SemaphoreType.DMA,
        pltpu.SemaphoreType.DMA,
        pltpu.SemaphoreType.DMA,
    ],
)
def _pack_sc(wt_hbm, t128_hbm, out_hbm, seg0, seg1, ov0, ov1,
             isem0, isem1, osem0, osem1):
    wid = lax.axis_index("s") * NUM_CORES + lax.axis_index("c")
    col0 = wid * CPT
    segs = (seg0, seg1)
    ovs = (ov0, ov1)
    isems = (isem0, isem1)
    osems = (osem0, osem1)
    lanes = lax.iota(jnp.int32, 16)
    lanes17 = lanes * STRIDE

    def in_copies(start, n, b):
        return [pltpu.make_async_copy(
            wt_hbm.at[:, pl.ds(start + 128 * t, 128)],
            segs[b].at[t], isems[b]) for t in range(n // 128)]

    def out_copy(start, n, b):
        return pltpu.make_async_copy(
            ovs[b].at[pl.ds(0, STRIDE * n)],
            out_hbm.at[pl.ds(STRIDE * start, STRIDE * n)], osems[b])

    def do_chunk(start, n, b):
        for c in in_copies(start, n, b):
            c.wait()
        seg, ov = segs[b], ovs[b]
        for t in range(n // 128):
            for sub in range(8):
                idx0 = lanes17 + (STRIDE * (128 * t + 16 * sub))
                for j in range(DIM):
                    v = seg[t, j, pl.ds(16 * sub, 16)]
                    plsc.store_scatter(ov, [idx0 + j], v)
        out_copy(start, n, b).start()

    for c in in_copies(col0, CC, 0) + in_copies(col0 + CC, CC, 1):
        c.start()

    def body(i, carry):
        for b in range(2):
            c = 2 * i + b
            @pl.when(c >= 2)
            def _():
                out_copy(col0 + CC * (c - 2), CC, b).wait()
            do_chunk(col0 + CC * c, CC, b)
            @pl.when(c + 2 < NCH)
            def _():
                for cp in in_copies(col0 + CC * (c + 2), CC, b):
                    cp.start()
        return carry

    lax.fori_loop(0, (NCH - 1) // 2, body, 0)

    is_last = wid == NW - 1

    @pl.when(is_last)
    def _():
        for cp in in_copies(TAIL0, 512, 1):
            cp.start()

    out_copy(col0 + CC * (NCH - 3), CC, 0).wait()
    do_chunk(col0 + CC * (NCH - 1), CC, 0)

    @pl.when(is_last)
    def _():
        pltpu.make_async_copy(t128_hbm, segs[0].at[0], isems[0]).start()
        out_copy(col0 + CC * (NCH - 2), CC, 1).wait()
        do_chunk(TAIL0, 512, 1)
        out_copy(col0 + CC * (NCH - 1), CC, 0).wait()
        pltpu.make_async_copy(t128_hbm, segs[0].at[0], isems[0]).wait()
        ov = ovs[0]
        for sub in range(8):
            idx0 = lanes17 + (STRIDE * 16 * sub)
            for j in range(DIM):
                v = seg0[0, j, pl.ds(16 * sub, 16)]
                plsc.store_scatter(ov, [idx0 + j], v)
        out_copy(TAIL1, 128, 0).start()
        out_copy(TAIL0, 512, 1).wait()
        out_copy(TAIL1, 128, 0).wait()

    @pl.when(jnp.logical_not(is_last))
    def _():
        out_copy(col0 + CC * (NCH - 2), CC, 1).wait()
        out_copy(col0 + CC * (NCH - 1), CC, 0).wait()


@functools.partial(
    pl.kernel,
    mesh=_MESH,
    out_type=jax.ShapeDtypeStruct((BATCH, DIM), jnp.float32),
    compiler_params=pltpu.CompilerParams(use_tc_tiling_on_sc=False),
    scratch_types=[
        pltpu.VMEM((IPW,), jnp.int32),
        pltpu.VMEM((BPW, DIM), jnp.float32),
    ] + [pltpu.VMEM((GROUP, STRIDE), jnp.float32) for _ in range(NBUF)]
      + [pltpu.SemaphoreType.DMA for _ in range(NBUF)],
)
def _embedding_bag_sc(idx_hbm, tbl_hbm, out_hbm, idx_v, out_v, *bufs):
    rows = bufs[:NBUF]
    sems = bufs[NBUF:]
    wid = lax.axis_index("s") * NUM_CORES + lax.axis_index("c")

    pltpu.sync_copy(idx_hbm.at[pl.ds(wid * IPW, IPW)], idx_v)

    def copies(g, b):
        base = GROUP * g
        cs = []
        off = 0
        while off < GROUP:
            n = min(SPLIT, GROUP - off)
            cs.append(pltpu.make_async_copy(
                tbl_hbm.at[idx_v.at[pl.ds(base + off, n)]],
                rows[b].at[pl.ds(off, n)], sems[b]))
            off += n
        return cs

    def start(g, b):
        for c in copies(g, b):
            c.start()

    def finish(g, b):
        for c in copies(g, b):
            c.wait()
        r = rows[b]
        for j in range(GROUP_BAGS):
            parts = []
            for c in range(5):
                base = BAG * j + 10 * c
                acc = r[base, pl.ds(0, DIM)]
                for k in range(base + 1, base + 10):
                    acc = acc + r[k, pl.ds(0, DIM)]
                parts.append(acc)
            total = (parts[0] + parts[1]) + (parts[2] + parts[3]) + parts[4]
            out_v[GROUP_BAGS * g + j] = total * jnp.float32(1.0 / BAG)

    for b in range(NBUF):
        start(b, b)

    def body(i, carry):
        for b in range(NBUF):
            g = NBUF * i + b
            finish(g, b)
            start(g + NBUF, b)
        return carry

    lax.fori_loop(0, GPW // NBUF - 1, body, 0)

    for b in range(NBUF):
        finish(GPW - NBUF + b, b)

    pltpu.sync_copy(out_v, out_hbm.at[pl.ds(wid * BPW, BPW)])


def kernel(input, weight):
    table = _pack_sc(weight.T, weight[NUM_EMB - 128:].T)
    table = table.reshape(NUM_EMB, STRIDE)
    return _embedding_bag_sc(input.astype(jnp.int32).reshape(-1), table)


# speedup vs baseline: 1.0850x; 1.0107x over previous
import functools

import jax
import jax.numpy as jnp
from jax import lax
from jax.experimental import pallas as pl
from jax.experimental.pallas import tpu as pltpu
from jax.experimental.pallas import tpu_sc as plsc

NUM_EMB = 1_000_000
DIM = 16
BATCH = 16384
BAG = 50

NUM_CORES = 2
NUM_SUBCORES = 16
NW = NUM_CORES * NUM_SUBCORES

BPW = BATCH // NW
IPW = BPW * BAG
GROUP_BAGS = 4
GROUP = GROUP_BAGS * BAG
SPLIT = 128
GPW = BPW // GROUP_BAGS
NBUF = 4

STRIDE = 16
CPT = 244 * 128
CC = 512
NCH = CPT // CC
TAIL0 = NW * CPT
TAIL1 = NUM_EMB - 128

_MESH = plsc.VectorSubcoreMesh(core_axis_name="c", subcore_axis_name="s")


@functools.partial(
    pl.kernel,
    mesh=_MESH,
    out_type=jax.ShapeDtypeStruct((NUM_EMB * STRIDE,), jnp.float32),
    compiler_params=pltpu.CompilerParams(use_tc_tiling_on_sc=True,
                                         needs_layout_passes=False),
    scratch_types=[
        pltpu.VMEM((CC // 128, DIM, 128), jnp.float32),
        pltpu.VMEM((CC // 128, DIM, 128), jnp.float32),
        pltpu.VMEM((CC * STRIDE,), jnp.float32),
        pltpu.VMEM((CC * STRIDE,), jnp.float32),
        pltpu.SemaphoreType.DMA,
        pltpu.SemaphoreType.DMA,
        pltpu.SemaphoreType.DMA,
        pltpu.SemaphoreType.DMA,
    ],
)
def _pack_sc(wt_hbm, t128_hbm, out_hbm, seg0, seg1, ov0, ov1,
             isem0, isem1, osem0, osem1):
    wid = lax.axis_index("s") * NUM_CORES + lax.axis_index("c")
    col0 = wid * CPT
    segs = (seg0, seg1)
    ovs = (ov0, ov1)
    isems = (isem0, isem1)
    osems = (osem0, osem1)
    lanes = lax.iota(jnp.int32, 16)
    lanes17 = lanes * STRIDE

    def in_copies(start, n, b):
        return [pltpu.make_async_copy(
            wt_hbm.at[:, pl.ds(start + 128 * t, 128)],
            segs[b].at[t], isems[b]) for t in range(n // 128)]

    def out_copy(start, n, b):
        return pltpu.make_async_copy(
            ovs[b].at[pl.ds(0, STRIDE * n)],
            out_hbm.at[pl.ds(STRIDE * start, STRIDE * n)], osems[b])

    def do_chunk(start, n, b):
        for c in in_copies(start, n, b):
            c.wait()
        seg, ov = segs[b], ovs[b]
        for t in range(n // 128):
            for sub in range(8):
                idx0 = lanes17 + (STRIDE * (128 * t + 16 * sub))
                vs = [seg[t, j, pl.ds(16 * sub, 16)] for j in range(DIM)]
                for j in range(DIM):
                    plsc.store_scatter(ov, [idx0 + j], vs[j])
        out_copy(start, n, b).start()

    for c in in_copies(col0, CC, 0) + in_copies(col0 + CC, CC, 1):
        c.start()

    def body(i, carry):
        for b in range(2):
            c = 2 * i + b
            @pl.when(c >= 2)
            def _():
                out_copy(col0 + CC * (c - 2), CC, b).wait()
            do_chunk(col0 + CC * c, CC, b)
            @pl.when(c + 2 < NCH)
            def _():
                for cp in in_copies(col0 + CC * (c + 2), CC, b):
                    cp.start()
        return carry

    lax.fori_loop(0, (NCH - 1) // 2, body, 0)

    is_last = wid == NW - 1

    @pl.when(is_last)
    def _():
        for cp in in_copies(TAIL0, 512, 1):
            cp.start()

    out_copy(col0 + CC * (NCH - 3), CC, 0).wait()
    do_chunk(col0 + CC * (NCH - 1), CC, 0)

    @pl.when(is_last)
    def _():
        pltpu.make_async_copy(t128_hbm, segs[0].at[0], isems[0]).start()
        out_copy(col0 + CC * (NCH - 2), CC, 1).wait()
        do_chunk(TAIL0, 512, 1)
        out_copy(col0 + CC * (NCH - 1), CC, 0).wait()
        pltpu.make_async_copy(t128_hbm, segs[0].at[0], isems[0]).wait()
        ov = ovs[0]
        for sub in range(8):
            idx0 = lanes17 + (STRIDE * 16 * sub)
            vs = [seg0[0, j, pl.ds(16 * sub, 16)] for j in range(DIM)]
            for j in range(DIM):
                plsc.store_scatter(ov, [idx0 + j], vs[j])
        out_copy(TAIL1, 128, 0).start()
        out_copy(TAIL0, 512, 1).wait()
        out_copy(TAIL1, 128, 0).wait()

    @pl.when(jnp.logical_not(is_last))
    def _():
        out_copy(col0 + CC * (NCH - 2), CC, 1).wait()
        out_copy(col0 + CC * (NCH - 1), CC, 0).wait()


@functools.partial(
    pl.kernel,
    mesh=_MESH,
    out_type=jax.ShapeDtypeStruct((BATCH, DIM), jnp.float32),
    compiler_params=pltpu.CompilerParams(use_tc_tiling_on_sc=False),
    scratch_types=[
        pltpu.VMEM((IPW,), jnp.int32),
        pltpu.VMEM((BPW, DIM), jnp.float32),
    ] + [pltpu.VMEM((GROUP, STRIDE), jnp.float32) for _ in range(NBUF)]
      + [pltpu.SemaphoreType.DMA for _ in range(NBUF)],
)
def _embedding_bag_sc(idx_hbm, tbl_hbm, out_hbm, idx_v, out_v, *bufs):
    rows = bufs[:NBUF]
    sems = bufs[NBUF:]
    wid = lax.axis_index("s") * NUM_CORES + lax.axis_index("c")

    pltpu.sync_copy(idx_hbm.at[pl.ds(wid * IPW, IPW)], idx_v)

    def copies(g, b):
        base = GROUP * g
        cs = []
        off = 0
        while off < GROUP:
            n = min(SPLIT, GROUP - off)
            cs.append(pltpu.make_async_copy(
                tbl_hbm.at[idx_v.at[pl.ds(base + off, n)]],
                rows[b].at[pl.ds(off, n)], sems[b]))
            off += n
        return cs

    def start(g, b):
        for c in copies(g, b):
            c.start()

    def finish(g, b):
        for c in copies(g, b):
            c.wait()
        r = rows[b]
        for j in range(GROUP_BAGS):
            parts = []
            for c in range(5):
                base = BAG * j + 10 * c
                acc = r[base, pl.ds(0, DIM)]
                for k in range(base + 1, base + 10):
                    acc = acc + r[k, pl.ds(0, DIM)]
                parts.append(acc)
            total = (parts[0] + parts[1]) + (parts[2] + parts[3]) + parts[4]
            out_v[GROUP_BAGS * g + j] = total * jnp.float32(1.0 / BAG)

    for b in range(NBUF):
        start(b, b)

    def body(i, carry):
        for b in range(NBUF):
            g = NBUF * i + b
            finish(g, b)
            start(g + NBUF, b)
        return carry

    lax.fori_loop(0, GPW // NBUF - 1, body, 0)

    for b in range(NBUF):
        finish(GPW - NBUF + b, b)

    pltpu.sync_copy(out_v, out_hbm.at[pl.ds(wid * BPW, BPW)])


def kernel(input, weight):
    table = _pack_sc(weight.T, weight[NUM_EMB - 128:].T)
    table = table.reshape(NUM_EMB, STRIDE)
    return _embedding_bag_sc(input.astype(jnp.int32).reshape(-1), table)
